# baseline (speedup 1.0000x reference)
import functools

import jax
import jax.numpy as jnp
from jax import lax
from jax.experimental import pallas as pl
from jax.experimental.pallas import tpu as pltpu
from jax.experimental.pallas import tpu_sc as plsc

_NC = 2
_NS = 16
_NW = _NC * _NS

_CHUNK = 400
_NBUF = 4


@functools.partial(jax.jit, static_argnums=(2, 3))
def _gather_flat(idx_flat, table, b_per_w, n_chunks):
    d = table.shape[1]
    mesh = plsc.VectorSubcoreMesh(core_axis_name="c", subcore_axis_name="s")
    n_outer = n_chunks // _NBUF

    @functools.partial(
        pl.kernel,
        mesh=mesh,
        out_type=jax.ShapeDtypeStruct((idx_flat.shape[0], 128), jnp.float32),
        scratch_types=[
            pltpu.VMEM((_NBUF * _CHUNK,), jnp.int32),
            pltpu.VMEM((_NBUF, _CHUNK, 64), jnp.float32),
        ]
        + [pltpu.SemaphoreType.DMA] * (3 * _NBUF),
        compiler_params=pltpu.CompilerParams(use_tc_tiling_on_sc=False),
    )
    def k(idx_hbm, table_hbm, out_hbm, idx_v, rows_v, *sems):
        idx_sem = sems[0:_NBUF]
        gat_sem = sems[_NBUF:2 * _NBUF]
        st_sem = sems[2 * _NBUF:3 * _NBUF]
        wid = lax.axis_index("s") * _NC + lax.axis_index("c")
        base = wid * b_per_w

        for b in range(_NBUF):
            pltpu.async_copy(
                idx_hbm.at[pl.ds(base + b * _CHUNK, _CHUNK)],
                idx_v.at[pl.ds(b * _CHUNK, _CHUNK)], idx_sem[b])

        def outer(g, _):
            for b in range(_NBUF):
                c = g * _NBUF + b
                off = base + c * _CHUNK
                idx_slice = idx_v.at[pl.ds(b * _CHUNK, _CHUNK)]
                dst = out_hbm.at[pl.ds(off, _CHUNK), pl.ds(0, 64)]
                pltpu.make_async_copy(
                    idx_hbm.at[pl.ds(off, _CHUNK)], idx_slice,
                    idx_sem[b]).wait()
                @pl.when(g > 0)
                def _():
                    pltpu.make_async_copy(rows_v.at[b], dst, st_sem[b]).wait()
                pltpu.async_copy(table_hbm.at[idx_slice], rows_v.at[b],
                                 gat_sem[b])
                pltpu.make_async_copy(table_hbm.at[idx_slice],
                                      rows_v.at[b], gat_sem[b]).wait()
                pltpu.async_copy(rows_v.at[b], dst, st_sem[b])
                @pl.when(c + _NBUF < n_chunks)
                def _():
                    pltpu.async_copy(
                        idx_hbm.at[pl.ds(off + _NBUF * _CHUNK, _CHUNK)],
                        idx_slice, idx_sem[b])
            return 0

        lax.fori_loop(0, n_outer, outer, 0)

        for b in range(_NBUF):
            pltpu.make_async_copy(
                rows_v.at[b],
                out_hbm.at[pl.ds(base, _CHUNK), pl.ds(0, 64)],
                st_sem[b]).wait()

    return k(idx_flat, table)


def kernel(indices, table):
    n, s = indices.shape
    b_total = n * s
    assert b_total % (_NW * _CHUNK * _NBUF) == 0
    b_per_w = b_total // _NW
    n_chunks = b_per_w // _CHUNK
    idx_flat = indices.reshape(b_total).astype(jnp.int32)
    out = _gather_flat(idx_flat, table, b_per_w, n_chunks)
    return out[:, : table.shape[1]].reshape(n, s, table.shape[1])


# speedup vs baseline: 1.0043x; 1.0043x over previous
import functools

import jax
import jax.numpy as jnp
from jax import lax
from jax.experimental import pallas as pl
from jax.experimental.pallas import tpu as pltpu
from jax.experimental.pallas import tpu_sc as plsc

_NC = 2
_NS = 16
_NW = _NC * _NS

_CHUNK = 400
_NBUF = 4


@functools.partial(jax.jit, static_argnums=(2, 3))
def _gather_flat(idx_flat, table, b_per_w, n_chunks):
    d = table.shape[1]
    mesh = plsc.VectorSubcoreMesh(core_axis_name="c", subcore_axis_name="s")
    n_outer = n_chunks // _NBUF

    @functools.partial(
        pl.kernel,
        mesh=mesh,
        out_type=jax.ShapeDtypeStruct((idx_flat.shape[0], 128), jnp.float32),
        scratch_types=[
            pltpu.VMEM((_NBUF * _CHUNK,), jnp.int32),
            pltpu.VMEM((_NBUF, _CHUNK, d), jnp.float32),
        ]
        + [pltpu.SemaphoreType.DMA] * (3 * _NBUF),
        compiler_params=pltpu.CompilerParams(use_tc_tiling_on_sc=False),
    )
    def k(idx_hbm, table_hbm, out_hbm, idx_v, rows_v, *sems):
        idx_sem = sems[0:_NBUF]
        gat_sem = sems[_NBUF:2 * _NBUF]
        st_sem = sems[2 * _NBUF:3 * _NBUF]
        wid = lax.axis_index("s") * _NC + lax.axis_index("c")
        base = wid * b_per_w

        def idx_slice(b):
            return idx_v.at[pl.ds(b * _CHUNK, _CHUNK)]

        def dst(off):
            return out_hbm.at[pl.ds(off, _CHUNK), pl.ds(0, d)]

        for b in range(_NBUF):
            pltpu.async_copy(
                idx_hbm.at[pl.ds(base + b * _CHUNK, _CHUNK)],
                idx_slice(b), idx_sem[b])
        for b in range(2):
            pltpu.make_async_copy(
                idx_hbm.at[pl.ds(base + b * _CHUNK, _CHUNK)],
                idx_slice(b), idx_sem[b]).wait()
            pltpu.async_copy(table_hbm.at[idx_slice(b)], rows_v.at[b],
                             gat_sem[b])

        def outer(g, _):
            for b in range(_NBUF):
                c = g * _NBUF + b
                off = base + c * _CHUNK
                b2 = (b + 2) % _NBUF
                pltpu.make_async_copy(table_hbm.at[idx_slice(b)],
                                      rows_v.at[b], gat_sem[b]).wait()
                pltpu.async_copy(rows_v.at[b], dst(off), st_sem[b])
                @pl.when(c + _NBUF < n_chunks)
                def _():
                    pltpu.async_copy(
                        idx_hbm.at[pl.ds(off + _NBUF * _CHUNK, _CHUNK)],
                        idx_slice(b), idx_sem[b])
                @pl.when(c + 2 < n_chunks)
                def _():
                    pltpu.make_async_copy(
                        idx_hbm.at[pl.ds(off, _CHUNK)], idx_slice(b2),
                        idx_sem[b2]).wait()
                    @pl.when(c >= 2)
                    def _():
                        pltpu.make_async_copy(
                            rows_v.at[b2], dst(off), st_sem[b2]).wait()
                    pltpu.async_copy(table_hbm.at[idx_slice(b2)],
                                     rows_v.at[b2], gat_sem[b2])
            return 0

        lax.fori_loop(0, n_outer, outer, 0)

        for b in range(_NBUF):
            pltpu.make_async_copy(
                rows_v.at[b], dst(base), st_sem[b]).wait()

    return k(idx_flat, table)


def kernel(indices, table):
    n, s = indices.shape
    b_total = n * s
    assert b_total % (_NW * _CHUNK * _NBUF) == 0
    b_per_w = b_total // _NW
    n_chunks = b_per_w // _CHUNK
    idx_flat = indices.reshape(b_total).astype(jnp.int32)
    out = _gather_flat(idx_flat, table, b_per_w, n_chunks)
    return out[:, : table.shape[1]].reshape(n, s, table.shape[1])
